# baseline (speedup 1.0000x reference)
import jax
import jax.numpy as jnp
from jax.experimental import pallas as pl
from jax.experimental.pallas import tpu as pltpu

_F32 = jnp.float32
_BF16 = jnp.bfloat16


def _fused_kernel(x_ref, t1_ref, b1_ref, t3a_ref, t3b_ref, t3c_ref, b3_ref,
                  w5_ref, b5_ref, w6_ref, b6_ref, wo_ref, bo_ref, out_ref):
    R = x_ref.shape[0]
    nb = out_ref.shape[0]

    x = x_ref[...]

    L1 = R - 4
    xc = jnp.concatenate([x[dy:dy + L1, :] for dy in range(5)], axis=1)
    a = jnp.dot(xc, t1_ref[...], preferred_element_type=_F32) + b1_ref[...]

    a = jnp.maximum(a[:, 0:128], a[:, 128:256])
    a = jnp.concatenate([a, jnp.zeros((4, 128), _F32)], axis=0)
    r = a.reshape(R // 2, 2, 128)
    a = jnp.maximum(r[:, 0, :], r[:, 1, :])
    a16 = jnp.maximum(a, 0.0).astype(_BF16)

    R2 = R // 2
    L3 = R2 - 4
    xa = jnp.concatenate([a16[0:L3, :], a16[1:1 + L3, :]], axis=1)
    xb = jnp.concatenate([a16[2:2 + L3, :], a16[3:3 + L3, :]], axis=1)
    a = (jnp.dot(xa, t3a_ref[...], preferred_element_type=_F32)
         + jnp.dot(xb, t3b_ref[...], preferred_element_type=_F32)
         + jnp.dot(a16[4:4 + L3, :], t3c_ref[...], preferred_element_type=_F32)
         + b3_ref[...])

    a = jnp.maximum(a[:, 0:128], a[:, 128:256])
    a = jnp.concatenate([a, jnp.zeros((4, 128), _F32)], axis=0)
    r = a.reshape(R2 // 2, 2, 128)
    a = jnp.maximum(r[:, 0, :], r[:, 1, :])
    a16 = jnp.maximum(a, 0.0).astype(_BF16)

    r3 = a16.reshape(nb, 8, 128)
    xc5 = jnp.concatenate([r3[:, dy, :] for dy in range(5)], axis=1)
    feats = jnp.maximum(
        jnp.dot(xc5, w5_ref[...], preferred_element_type=_F32)
        + b5_ref[...], 0.0)

    h = jnp.maximum(
        jnp.dot(feats.astype(_BF16), w6_ref[...], preferred_element_type=_F32)
        + b6_ref[...], 0.0)
    out_ref[...] = (jnp.dot(h.astype(_BF16), wo_ref[...],
                            preferred_element_type=_F32)
                    + bo_ref[...]).astype(out_ref.dtype)


def kernel(img, T1, B1, T3, B3, T5, B5, W6p, B6p, WOp, BOp, *, block_batch=64):
    B = img.shape[0]
    nb = block_batch
    Bp = ((B + nb - 1) // nb) * nb

    x = img.reshape(B, 28, 28)
    x = jnp.pad(x, ((0, Bp - B), (2, 2), (0, 0))).astype(_BF16)
    x = x.reshape(Bp * 32, 28)

    T1f = T1[:, 2:30, :].reshape(140, 256)
    T3a = T3[0:2].reshape(256, 256)
    T3b = T3[2:4].reshape(256, 256)
    T3c = T3[4]
    W5f = T5.reshape(640, 128)

    weights = (T1f, B1, T3a, T3b, T3c, B3, W5f, B5, W6p, B6p, WOp, BOp)
    w_specs = [pl.BlockSpec(t.shape, lambda i, n=t.ndim: (0,) * n)
               for t in weights]
    grid = (Bp // nb,)

    out = pl.pallas_call(
        _fused_kernel,
        out_shape=jax.ShapeDtypeStruct((Bp, 128), jnp.float32),
        grid=grid,
        in_specs=[pl.BlockSpec((nb * 32, 28), lambda i: (i, 0))] + w_specs,
        out_specs=pl.BlockSpec((nb, 128), lambda i: (i, 0)),
        compiler_params=pltpu.CompilerParams(
            dimension_semantics=("parallel",),
            vmem_limit_bytes=64 * 1024 * 1024),
    )(x, *weights)

    return out[:B, :10]


# speedup vs baseline: 3.4108x; 3.4108x over previous
import jax
import jax.numpy as jnp
from jax.experimental import pallas as pl
from jax.experimental.pallas import tpu as pltpu

_F32 = jnp.float32
_BF16 = jnp.bfloat16


def _fused_kernel(x_ref, t1_ref, b1_ref, w3a_ref, w3b_ref, w3c_ref,
                  w3d_ref, w3e_ref, w3f_ref, b3_ref,
                  w5_ref, b5_ref, w6_ref, b6_ref, wo_ref, bo_ref, out_ref):
    M8 = x_ref.shape[0]
    nb = out_ref.shape[0]

    x = x_ref[...].astype(_BF16)
    L = M8 - 1
    t1 = t1_ref[...]

    xp0 = jnp.concatenate([x[0:L, 0:112], x[1:1 + L, 0:28]], axis=1)
    xp1 = jnp.concatenate([x[0:L, 28:112], x[1:1 + L, 0:56]], axis=1)
    xp2 = jnp.concatenate([x[0:L, 56:112], x[1:1 + L, 0:84]], axis=1)
    xp3 = jnp.concatenate([x[0:L, 84:112], x[1:1 + L, 0:112]], axis=1)
    c10 = jnp.dot(xp0, t1, preferred_element_type=_F32)
    c11 = jnp.dot(xp1, t1, preferred_element_type=_F32)
    c12 = jnp.dot(xp2, t1, preferred_element_type=_F32)
    c13 = jnp.dot(xp3, t1, preferred_element_type=_F32)

    b1 = b1_ref[...]
    pe = jnp.maximum(c10, c11) + b1
    po = jnp.maximum(c12, c13) + b1
    se = jnp.maximum(jnp.maximum(pe[:, 0:128], pe[:, 128:256]), 0.0)
    so = jnp.maximum(jnp.maximum(po[:, 0:128], po[:, 128:256]), 0.0)
    se = se.astype(_BF16)
    so = so.astype(_BF16)
    eo = jnp.concatenate([se, so], axis=1)

    L3 = M8 - 3
    c3e = (jnp.dot(eo[0:L3, :], w3a_ref[...], preferred_element_type=_F32)
           + jnp.dot(eo[1:1 + L3, :], w3b_ref[...], preferred_element_type=_F32)
           + jnp.dot(se[2:2 + L3, :], w3c_ref[...], preferred_element_type=_F32))
    c3o = (jnp.dot(so[0:L3, :], w3f_ref[...], preferred_element_type=_F32)
           + jnp.dot(eo[1:1 + L3, :], w3d_ref[...], preferred_element_type=_F32)
           + jnp.dot(eo[2:2 + L3, :], w3e_ref[...], preferred_element_type=_F32))

    b3 = b3_ref[...]
    p4 = jnp.maximum(c3e, c3o) + b3
    a4 = jnp.maximum(jnp.maximum(p4[:, 0:128], p4[:, 128:256]), 0.0)
    a4 = a4.astype(_BF16)

    a4 = jnp.concatenate([a4, jnp.zeros((3, 128), _BF16)], axis=0)
    r3 = a4.reshape(nb, 8, 128)
    xc5 = jnp.concatenate([r3[:, dy, :] for dy in range(5)], axis=1)
    feats = jnp.maximum(
        jnp.dot(xc5, w5_ref[...], preferred_element_type=_F32)
        + b5_ref[...], 0.0)

    h = jnp.maximum(
        jnp.dot(feats.astype(_BF16), w6_ref[...], preferred_element_type=_F32)
        + b6_ref[...], 0.0)
    out_ref[...] = (jnp.dot(h.astype(_BF16), wo_ref[...],
                            preferred_element_type=_F32)
                    + bo_ref[...]).astype(out_ref.dtype)


def kernel(img, T1, B1, T3, B3, T5, B5, W6p, B6p, WOp, BOp, *, block_batch=64):
    B = img.shape[0]
    nb = block_batch
    Bp = ((B + nb - 1) // nb) * nb

    x = img.reshape(B, 28, 28)
    x = jnp.pad(x, ((0, Bp - B), (2, 2), (0, 0)))
    x = x.reshape(Bp * 8, 112)

    T1f = T1[:, 2:30, :].reshape(140, 256)
    W3a = T3[0:2].reshape(256, 256)
    W3b = T3[2:4].reshape(256, 256)
    W3c = T3[4]
    W3f = T3[0]
    W3d = T3[1:3].reshape(256, 256)
    W3e = T3[3:5].reshape(256, 256)
    W5f = T5.reshape(640, 128)

    weights = (T1f, B1, W3a, W3b, W3c, W3d, W3e, W3f, B3,
               W5f, B5, W6p, B6p, WOp, BOp)
    w_specs = [pl.BlockSpec(t.shape, lambda i, n=t.ndim: (0,) * n)
               for t in weights]
    grid = (Bp // nb,)

    out = pl.pallas_call(
        _fused_kernel,
        out_shape=jax.ShapeDtypeStruct((Bp, 128), jnp.float32),
        grid=grid,
        in_specs=[pl.BlockSpec((nb * 8, 112), lambda i: (i, 0))] + w_specs,
        out_specs=pl.BlockSpec((nb, 128), lambda i: (i, 0)),
        compiler_params=pltpu.CompilerParams(
            dimension_semantics=("parallel",),
            vmem_limit_bytes=64 * 1024 * 1024),
    )(x, *weights)

    return out[:B, :10]


# speedup vs baseline: 3.7876x; 1.1105x over previous
import jax
import jax.numpy as jnp
from jax.experimental import pallas as pl
from jax.experimental.pallas import tpu as pltpu

_F32 = jnp.float32
_BF16 = jnp.bfloat16


def _fused_kernel(x_ref, t1_ref, b1_ref, w3a_ref, w3b_ref, w3c_ref,
                  w3d_ref, w3e_ref, w3f_ref, b3_ref,
                  w5_ref, b5_ref, w6_ref, b6_ref, wo_ref, bo_ref, out_ref):
    M8 = x_ref.shape[0]
    nb = out_ref.shape[0]

    x = x_ref[...].astype(_BF16)
    L = M8 - 1
    t1 = t1_ref[...]

    xp0 = jnp.concatenate([x[0:L, 0:112], x[1:1 + L, 0:28]], axis=1)
    xp1 = jnp.concatenate([x[0:L, 28:112], x[1:1 + L, 0:56]], axis=1)
    xp2 = jnp.concatenate([x[0:L, 56:112], x[1:1 + L, 0:84]], axis=1)
    xp3 = jnp.concatenate([x[0:L, 84:112], x[1:1 + L, 0:112]], axis=1)
    c10 = jnp.dot(xp0, t1, preferred_element_type=_F32)
    c11 = jnp.dot(xp1, t1, preferred_element_type=_F32)
    c12 = jnp.dot(xp2, t1, preferred_element_type=_F32)
    c13 = jnp.dot(xp3, t1, preferred_element_type=_F32)

    b1 = b1_ref[...]
    pe = jnp.maximum(c10, c11) + b1
    po = jnp.maximum(c12, c13) + b1
    se = jnp.maximum(jnp.maximum(pe[:, 0:128], pe[:, 128:256]), 0.0)
    so = jnp.maximum(jnp.maximum(po[:, 0:128], po[:, 128:256]), 0.0)
    se = se.astype(_BF16)
    so = so.astype(_BF16)
    eo = jnp.concatenate([se, so], axis=1)

    L3 = M8 - 3
    c3e = (jnp.dot(eo[0:L3, :], w3a_ref[...], preferred_element_type=_F32)
           + jnp.dot(eo[1:1 + L3, :], w3b_ref[...], preferred_element_type=_F32)
           + jnp.dot(se[2:2 + L3, :], w3c_ref[...], preferred_element_type=_F32))
    c3o = (jnp.dot(so[0:L3, :], w3f_ref[...], preferred_element_type=_F32)
           + jnp.dot(eo[1:1 + L3, :], w3d_ref[...], preferred_element_type=_F32)
           + jnp.dot(eo[2:2 + L3, :], w3e_ref[...], preferred_element_type=_F32))

    b3 = b3_ref[...]
    p4 = jnp.maximum(c3e, c3o) + b3
    a4 = jnp.maximum(jnp.maximum(p4[:, 0:128], p4[:, 128:256]), 0.0)
    a4 = a4.astype(_BF16)

    a4 = jnp.concatenate([a4, jnp.zeros((3, 128), _BF16)], axis=0)
    r3 = a4.reshape(nb, 8, 128)
    xc5 = jnp.concatenate([r3[:, dy, :] for dy in range(5)], axis=1)
    feats = jnp.maximum(
        jnp.dot(xc5, w5_ref[...], preferred_element_type=_F32)
        + b5_ref[...], 0.0)

    h = jnp.maximum(
        jnp.dot(feats.astype(_BF16), w6_ref[...], preferred_element_type=_F32)
        + b6_ref[...], 0.0)
    out_ref[...] = (jnp.dot(h.astype(_BF16), wo_ref[...],
                            preferred_element_type=_F32)
                    + bo_ref[...]).astype(out_ref.dtype)


def kernel(img, T1, B1, T3, B3, T5, B5, W6p, B6p, WOp, BOp, *, block_batch=128):
    B = img.shape[0]
    nb = block_batch
    Bp = ((B + nb - 1) // nb) * nb

    x = img.reshape(B, 28, 28)
    x = jnp.pad(x, ((0, Bp - B), (2, 2), (0, 0)))
    x = x.reshape(Bp * 8, 112)

    T1f = T1[:, 2:30, :].reshape(140, 256)
    W3a = T3[0:2].reshape(256, 256)
    W3b = T3[2:4].reshape(256, 256)
    W3c = T3[4]
    W3f = T3[0]
    W3d = T3[1:3].reshape(256, 256)
    W3e = T3[3:5].reshape(256, 256)
    W5f = T5.reshape(640, 128)

    weights = (T1f, B1, W3a, W3b, W3c, W3d, W3e, W3f, B3,
               W5f, B5, W6p, B6p, WOp, BOp)
    w_specs = [pl.BlockSpec(t.shape, lambda i, n=t.ndim: (0,) * n)
               for t in weights]
    grid = (Bp // nb,)

    out = pl.pallas_call(
        _fused_kernel,
        out_shape=jax.ShapeDtypeStruct((Bp, 128), jnp.float32),
        grid=grid,
        in_specs=[pl.BlockSpec((nb * 8, 112), lambda i: (i, 0))] + w_specs,
        out_specs=pl.BlockSpec((nb, 128), lambda i: (i, 0)),
        compiler_params=pltpu.CompilerParams(
            dimension_semantics=("parallel",),
            vmem_limit_bytes=64 * 1024 * 1024),
    )(x, *weights)

    return out[:B, :10]


# speedup vs baseline: 3.9693x; 1.0480x over previous
import jax
import jax.numpy as jnp
from jax.experimental import pallas as pl
from jax.experimental.pallas import tpu as pltpu

_F32 = jnp.float32
_BF16 = jnp.bfloat16


def _fused_kernel(x_ref, t1_ref, b1_ref, w3a_ref, w3b_ref, w3c_ref,
                  w3d_ref, w3e_ref, w3f_ref, b3_ref,
                  w5_ref, b5_ref, w6_ref, b6_ref, wo_ref, bo_ref, out_ref):
    M8 = x_ref.shape[0]
    nb = out_ref.shape[0]

    x = x_ref[...].astype(_BF16)
    L = M8 - 1
    t1 = t1_ref[...]

    xp0 = jnp.concatenate([x[0:L, 0:112], x[1:1 + L, 0:28]], axis=1)
    xp1 = jnp.concatenate([x[0:L, 28:112], x[1:1 + L, 0:56]], axis=1)
    xp2 = jnp.concatenate([x[0:L, 56:112], x[1:1 + L, 0:84]], axis=1)
    xp3 = jnp.concatenate([x[0:L, 84:112], x[1:1 + L, 0:112]], axis=1)
    c10 = jnp.dot(xp0, t1, preferred_element_type=_F32)
    c11 = jnp.dot(xp1, t1, preferred_element_type=_F32)
    c12 = jnp.dot(xp2, t1, preferred_element_type=_F32)
    c13 = jnp.dot(xp3, t1, preferred_element_type=_F32)

    b1 = b1_ref[...]
    pe = jnp.maximum(c10, c11) + b1
    po = jnp.maximum(c12, c13) + b1
    se = jnp.maximum(jnp.maximum(pe[:, 0:128], pe[:, 128:256]), 0.0)
    so = jnp.maximum(jnp.maximum(po[:, 0:128], po[:, 128:256]), 0.0)
    se = se.astype(_BF16)
    so = so.astype(_BF16)
    eo = jnp.concatenate([se, so], axis=1)

    L3 = M8 - 3
    c3e = (jnp.dot(eo[0:L3, :], w3a_ref[...], preferred_element_type=_F32)
           + jnp.dot(eo[1:1 + L3, :], w3b_ref[...], preferred_element_type=_F32)
           + jnp.dot(se[2:2 + L3, :], w3c_ref[...], preferred_element_type=_F32))
    c3o = (jnp.dot(so[0:L3, :], w3f_ref[...], preferred_element_type=_F32)
           + jnp.dot(eo[1:1 + L3, :], w3d_ref[...], preferred_element_type=_F32)
           + jnp.dot(eo[2:2 + L3, :], w3e_ref[...], preferred_element_type=_F32))

    b3 = b3_ref[...]
    p4 = jnp.maximum(c3e, c3o) + b3
    a4 = jnp.maximum(jnp.maximum(p4[:, 0:128], p4[:, 128:256]), 0.0)
    a4 = a4.astype(_BF16)

    a4 = jnp.concatenate([a4, jnp.zeros((3, 128), _BF16)], axis=0)
    r3 = a4.reshape(nb, 8, 128)
    xc5 = jnp.concatenate([r3[:, dy, :] for dy in range(5)], axis=1)
    feats = jnp.maximum(
        jnp.dot(xc5, w5_ref[...], preferred_element_type=_F32)
        + b5_ref[...], 0.0)

    h = jnp.maximum(
        jnp.dot(feats.astype(_BF16), w6_ref[...], preferred_element_type=_F32)
        + b6_ref[...], 0.0)
    out_ref[...] = (jnp.dot(h.astype(_BF16), wo_ref[...],
                            preferred_element_type=_F32)
                    + bo_ref[...]).astype(out_ref.dtype)


def kernel(img, T1, B1, T3, B3, T5, B5, W6p, B6p, WOp, BOp, *, block_batch=256):
    B = img.shape[0]
    nb = block_batch
    Bp = ((B + nb - 1) // nb) * nb

    x = img.reshape(B, 28, 28)
    x = jnp.pad(x, ((0, Bp - B), (2, 2), (0, 0)))
    x = x.reshape(Bp * 8, 112)

    T1f = T1[:, 2:30, :].reshape(140, 256)
    W3a = T3[0:2].reshape(256, 256)
    W3b = T3[2:4].reshape(256, 256)
    W3c = T3[4]
    W3f = T3[0]
    W3d = T3[1:3].reshape(256, 256)
    W3e = T3[3:5].reshape(256, 256)
    W5f = T5.reshape(640, 128)

    weights = (T1f, B1, W3a, W3b, W3c, W3d, W3e, W3f, B3,
               W5f, B5, W6p, B6p, WOp, BOp)
    w_specs = [pl.BlockSpec(t.shape, lambda i, n=t.ndim: (0,) * n)
               for t in weights]
    grid = (Bp // nb,)

    out = pl.pallas_call(
        _fused_kernel,
        out_shape=jax.ShapeDtypeStruct((Bp, 128), jnp.float32),
        grid=grid,
        in_specs=[pl.BlockSpec((nb * 8, 112), lambda i: (i, 0))] + w_specs,
        out_specs=pl.BlockSpec((nb, 128), lambda i: (i, 0)),
        compiler_params=pltpu.CompilerParams(
            dimension_semantics=("parallel",),
            vmem_limit_bytes=64 * 1024 * 1024),
    )(x, *weights)

    return out[:B, :10]
